# Initial kernel scaffold; baseline (speedup 1.0000x reference)
import jax
import jax.numpy as jnp
from jax.experimental import pallas as pl


def kernel(x, batch):
    raise NotImplementedError("write your pallas kernel here")


# speedup vs baseline: 4.8304x; 4.8304x over previous
import functools

import jax
import jax.numpy as jnp
from jax import lax
from jax.experimental import pallas as pl
from jax.experimental.pallas import tpu as pltpu
from jax.experimental.pallas import tpu_sc as plsc

N = 100000
D = 128
G = 512

NC = 2
NS = 16
HALF = D // NC
R = 512
NFULL = N // R
TAIL = N - NFULL * R
SEG_PER_TILE = G // NS

_mesh = plsc.VectorSubcoreMesh(core_axis_name="c", subcore_axis_name="s")


@functools.partial(
    pl.kernel,
    out_type=jax.ShapeDtypeStruct((G, D), jnp.float32),
    mesh=_mesh,
    compiler_params=pltpu.CompilerParams(use_tc_tiling_on_sc=False),
    scratch_types=[
        pltpu.VMEM((R, HALF), jnp.float32),
        pltpu.VMEM((128,), jnp.int32),
        pltpu.VMEM((32,), jnp.int32),
        pltpu.VMEM((128, 16), jnp.float32),
        pltpu.VMEM((SEG_PER_TILE, HALF), jnp.float32),
        pltpu.VMEM((SEG_PER_TILE, 16), jnp.float32),
        pltpu.VMEM_SHARED((G, HALF), jnp.float32),
        pltpu.VMEM_SHARED((G, 16), jnp.float32),
    ],
)
def _pool_kernel(x_hbm, b_hbm, z_hbm, ones_hbm, out_hbm,
                 xb, idx, idx_t, onesv, accv, cntv, acc_sh, cnt_sh):
    c = lax.axis_index("c")
    s = lax.axis_index("s")
    col0 = c * HALF
    seg0 = s * SEG_PER_TILE

    pltpu.sync_copy(z_hbm, acc_sh.at[pl.ds(seg0, SEG_PER_TILE)])
    pltpu.sync_copy(z_hbm.at[:, pl.ds(0, 16)],
                    cnt_sh.at[pl.ds(seg0, SEG_PER_TILE)])
    pltpu.sync_copy(ones_hbm, onesv)
    plsc.subcore_barrier()

    nblk = jnp.where(s < NFULL % NS, NFULL // NS + 1, NFULL // NS)

    def blk_body(k, carry):
        r0 = (s + NS * k) * R
        pltpu.sync_copy(x_hbm.at[pl.ds(r0, R), pl.ds(col0, HALF)], xb)
        for m in range(R // 128):
            pltpu.sync_copy(b_hbm.at[pl.ds(r0 + 128 * m, 128)], idx)
            pltpu.sync_copy(xb.at[pl.ds(128 * m, 128)], acc_sh.at[idx],
                            add=True)
            pltpu.sync_copy(onesv, cnt_sh.at[idx], add=True)
        return carry

    lax.fori_loop(0, nblk, blk_body, 0)

    @pl.when(s == NS - 1)
    def _tail():
        r0 = NFULL * R
        pltpu.sync_copy(x_hbm.at[pl.ds(r0, TAIL), pl.ds(col0, HALF)],
                        xb.at[pl.ds(0, TAIL)])
        pltpu.sync_copy(b_hbm.at[pl.ds(r0, 128)], idx)
        pltpu.sync_copy(xb.at[pl.ds(0, 128)], acc_sh.at[idx], add=True)
        pltpu.sync_copy(onesv, cnt_sh.at[idx], add=True)
        pltpu.sync_copy(b_hbm.at[pl.ds(r0 + 128, 32)], idx_t)
        pltpu.sync_copy(xb.at[pl.ds(128, 32)], acc_sh.at[idx_t], add=True)
        pltpu.sync_copy(onesv.at[pl.ds(0, 32)], cnt_sh.at[idx_t], add=True)

    plsc.subcore_barrier()

    pltpu.sync_copy(acc_sh.at[pl.ds(seg0, SEG_PER_TILE)], accv)
    pltpu.sync_copy(cnt_sh.at[pl.ds(seg0, SEG_PER_TILE)], cntv)
    for i in range(SEG_PER_TILE):
        inv = 1.0 / jnp.maximum(cntv[i, :], 1.0)
        for j in range(HALF // 16):
            accv[i, pl.ds(16 * j, 16)] = accv[i, pl.ds(16 * j, 16)] * inv
    pltpu.sync_copy(accv,
                    out_hbm.at[pl.ds(seg0, SEG_PER_TILE), pl.ds(col0, HALF)])


def kernel(x, batch):
    zeros = jnp.zeros((SEG_PER_TILE, HALF), jnp.float32)
    ones = jnp.ones((128, 16), jnp.float32)
    return _pool_kernel(x, batch, zeros, ones)


# speedup vs baseline: 6.7687x; 1.4013x over previous
import functools

import jax
import jax.numpy as jnp
from jax import lax
from jax.experimental import pallas as pl
from jax.experimental.pallas import tpu as pltpu
from jax.experimental.pallas import tpu_sc as plsc

N = 100000
D = 128
G = 512

NC = 2
NS = 16
HALF = D // NC
R = 512
NFULL = N // R
TAIL = N - NFULL * R
NB = NFULL // NS
EXTRA = NFULL - NB * NS
SEG_PER_TILE = G // NS
CH = R // 128

_mesh = plsc.VectorSubcoreMesh(core_axis_name="c", subcore_axis_name="s")


@functools.partial(
    pl.kernel,
    out_type=jax.ShapeDtypeStruct((G, D), jnp.float32),
    mesh=_mesh,
    compiler_params=pltpu.CompilerParams(use_tc_tiling_on_sc=False),
    scratch_types=[
        pltpu.VMEM((2, R, HALF), jnp.float32),
        pltpu.VMEM((2, CH, 128), jnp.int32),
        pltpu.VMEM((32,), jnp.int32),
        pltpu.VMEM((128, 16), jnp.float32),
        pltpu.VMEM((SEG_PER_TILE, HALF), jnp.float32),
        pltpu.VMEM((SEG_PER_TILE, 16), jnp.float32),
        pltpu.VMEM_SHARED((G, HALF), jnp.float32),
        pltpu.VMEM_SHARED((G, 16), jnp.float32),
        pltpu.SemaphoreType.DMA,
        pltpu.SemaphoreType.DMA,
        pltpu.SemaphoreType.DMA,
        pltpu.SemaphoreType.DMA,
    ],
)
def _pool_kernel(x_hbm, b_hbm, z_hbm, ones_hbm, out_hbm,
                 xb, idxb, idx_t, onesv, accv, cntv, acc_sh, cnt_sh,
                 sx0, sx1, si0, si1):
    c = lax.axis_index("c")
    s = lax.axis_index("s")
    col0 = c * HALF
    seg0 = s * SEG_PER_TILE
    sx = (sx0, sx1)
    si = (si0, si1)

    def start_load(kblk, b):
        r0 = (s + NS * kblk) * R
        pltpu.async_copy(x_hbm.at[pl.ds(r0, R), pl.ds(col0, HALF)],
                         xb.at[b], sx[b])
        for m in range(CH):
            pltpu.async_copy(b_hbm.at[pl.ds(r0 + 128 * m, 128)],
                             idxb.at[b, m], si[b])

    def wait_load(b):
        pltpu.make_async_copy(x_hbm.at[pl.ds(0, R), pl.ds(0, HALF)],
                              xb.at[b], sx[b]).wait()
        for m in range(CH):
            pltpu.make_async_copy(b_hbm.at[pl.ds(0, 128)],
                                  idxb.at[b, m], si[b]).wait()

    def scatter_buf(b):
        for m in range(CH):
            pltpu.sync_copy(xb.at[b, pl.ds(128 * m, 128)],
                            acc_sh.at[idxb.at[b, m]], add=True)
            pltpu.sync_copy(onesv, cnt_sh.at[idxb.at[b, m]], add=True)

    start_load(0, 0)
    start_load(1, 1)
    pltpu.sync_copy(z_hbm, acc_sh.at[pl.ds(seg0, SEG_PER_TILE)])
    pltpu.sync_copy(z_hbm.at[:, pl.ds(0, 16)],
                    cnt_sh.at[pl.ds(seg0, SEG_PER_TILE)])
    pltpu.sync_copy(ones_hbm, onesv)
    plsc.subcore_barrier()

    def blk_body(g, carry):
        for b in range(2):
            wait_load(b)
            scatter_buf(b)
            start_load(2 * g + b + 2, b)
        return carry

    lax.fori_loop(0, NB // 2 - 1, blk_body, 0)
    for b in range(2):
        wait_load(b)
        scatter_buf(b)

    @pl.when(s < EXTRA)
    def _extra():
        r0 = (NB * NS + s) * R
        pltpu.sync_copy(x_hbm.at[pl.ds(r0, R), pl.ds(col0, HALF)], xb.at[0])
        for m in range(CH):
            pltpu.sync_copy(b_hbm.at[pl.ds(r0 + 128 * m, 128)], idxb.at[0, m])
            pltpu.sync_copy(xb.at[0, pl.ds(128 * m, 128)],
                            acc_sh.at[idxb.at[0, m]], add=True)
            pltpu.sync_copy(onesv, cnt_sh.at[idxb.at[0, m]], add=True)

    @pl.when(s == EXTRA)
    def _tail():
        r0 = NFULL * R
        pltpu.sync_copy(x_hbm.at[pl.ds(r0, TAIL), pl.ds(col0, HALF)],
                        xb.at[0, pl.ds(0, TAIL)])
        pltpu.sync_copy(b_hbm.at[pl.ds(r0, 128)], idxb.at[0, 0])
        pltpu.sync_copy(xb.at[0, pl.ds(0, 128)], acc_sh.at[idxb.at[0, 0]],
                        add=True)
        pltpu.sync_copy(onesv, cnt_sh.at[idxb.at[0, 0]], add=True)
        pltpu.sync_copy(b_hbm.at[pl.ds(r0 + 128, 32)], idx_t)
        pltpu.sync_copy(xb.at[0, pl.ds(128, 32)], acc_sh.at[idx_t], add=True)
        pltpu.sync_copy(onesv.at[pl.ds(0, 32)], cnt_sh.at[idx_t], add=True)

    plsc.subcore_barrier()

    pltpu.sync_copy(acc_sh.at[pl.ds(seg0, SEG_PER_TILE)], accv)
    pltpu.sync_copy(cnt_sh.at[pl.ds(seg0, SEG_PER_TILE)], cntv)
    for i in range(SEG_PER_TILE):
        inv = 1.0 / jnp.maximum(cntv[i, :], 1.0)
        for j in range(HALF // 16):
            accv[i, pl.ds(16 * j, 16)] = accv[i, pl.ds(16 * j, 16)] * inv
    pltpu.sync_copy(accv,
                    out_hbm.at[pl.ds(seg0, SEG_PER_TILE), pl.ds(col0, HALF)])


def kernel(x, batch):
    zeros = jnp.zeros((SEG_PER_TILE, HALF), jnp.float32)
    ones = jnp.ones((128, 16), jnp.float32)
    return _pool_kernel(x, batch, zeros, ones)


# speedup vs baseline: 6.7756x; 1.0010x over previous
import functools

import jax
import jax.numpy as jnp
from jax import lax
from jax.experimental import pallas as pl
from jax.experimental.pallas import tpu as pltpu
from jax.experimental.pallas import tpu_sc as plsc

N = 100000
D = 128
G = 512

NC = 2
NS = 16
HALF = D // NC
R = 512
NFULL = N // R
TAIL = N - NFULL * R
NB = NFULL // NS
EXTRA = NFULL - NB * NS
SEG_PER_TILE = G // NS
CH = R // 128

_mesh = plsc.VectorSubcoreMesh(core_axis_name="c", subcore_axis_name="s")


@functools.partial(
    pl.kernel,
    out_type=jax.ShapeDtypeStruct((G, D), jnp.float32),
    mesh=_mesh,
    compiler_params=pltpu.CompilerParams(use_tc_tiling_on_sc=False),
    scratch_types=[
        pltpu.VMEM((2, R, HALF), jnp.float32),
        pltpu.VMEM((2, CH, 128), jnp.int32),
        pltpu.VMEM((32,), jnp.int32),
        pltpu.VMEM((128, 16), jnp.float32),
        pltpu.VMEM((SEG_PER_TILE, HALF), jnp.float32),
        pltpu.VMEM((SEG_PER_TILE, 16), jnp.float32),
        pltpu.VMEM_SHARED((G, HALF), jnp.float32),
        pltpu.VMEM_SHARED((G, 16), jnp.float32),
        pltpu.SemaphoreType.DMA,
        pltpu.SemaphoreType.DMA,
        pltpu.SemaphoreType.DMA,
        pltpu.SemaphoreType.DMA,
        pltpu.SemaphoreType.DMA,
    ],
)
def _pool_kernel(x_hbm, b_hbm, z_hbm, ones_hbm, out_hbm,
                 xb, idxb, idx_t, onesv, accv, cntv, acc_sh, cnt_sh,
                 sx0, sx1, si0, si1, ssc):
    c = lax.axis_index("c")
    s = lax.axis_index("s")
    col0 = c * HALF
    seg0 = s * SEG_PER_TILE
    sx = (sx0, sx1)
    si = (si0, si1)

    def start_load(kblk, b):
        r0 = (s + NS * kblk) * R
        pltpu.async_copy(x_hbm.at[pl.ds(r0, R), pl.ds(col0, HALF)],
                         xb.at[b], sx[b])
        for m in range(CH):
            pltpu.async_copy(b_hbm.at[pl.ds(r0 + 128 * m, 128)],
                             idxb.at[b, m], si[b])

    def wait_load(b):
        pltpu.make_async_copy(x_hbm.at[pl.ds(0, R), pl.ds(0, HALF)],
                              xb.at[b], sx[b]).wait()
        for m in range(CH):
            pltpu.make_async_copy(b_hbm.at[pl.ds(0, 128)],
                                  idxb.at[b, m], si[b]).wait()

    def scatter_buf(b):
        handles = []
        for m in range(CH):
            handles.append(pltpu.async_copy(
                xb.at[b, pl.ds(128 * m, 128)],
                acc_sh.at[idxb.at[b, m]], ssc, add=True))
            handles.append(pltpu.async_copy(
                onesv, cnt_sh.at[idxb.at[b, m]], ssc, add=True))
        for h in handles:
            h.wait()

    start_load(0, 0)
    start_load(1, 1)
    pltpu.sync_copy(z_hbm, acc_sh.at[pl.ds(seg0, SEG_PER_TILE)])
    pltpu.sync_copy(z_hbm.at[:, pl.ds(0, 16)],
                    cnt_sh.at[pl.ds(seg0, SEG_PER_TILE)])
    pltpu.sync_copy(ones_hbm, onesv)
    plsc.subcore_barrier()

    def blk_body(g, carry):
        for b in range(2):
            wait_load(b)
            scatter_buf(b)
            start_load(2 * g + b + 2, b)
        return carry

    lax.fori_loop(0, NB // 2 - 1, blk_body, 0)
    for b in range(2):
        wait_load(b)
        scatter_buf(b)

    @pl.when(s < EXTRA)
    def _extra():
        r0 = (NB * NS + s) * R
        pltpu.sync_copy(x_hbm.at[pl.ds(r0, R), pl.ds(col0, HALF)], xb.at[0])
        for m in range(CH):
            pltpu.sync_copy(b_hbm.at[pl.ds(r0 + 128 * m, 128)], idxb.at[0, m])
        scatter_buf(0)

    @pl.when(s == EXTRA)
    def _tail():
        r0 = NFULL * R
        pltpu.sync_copy(x_hbm.at[pl.ds(r0, TAIL), pl.ds(col0, HALF)],
                        xb.at[0, pl.ds(0, TAIL)])
        pltpu.sync_copy(b_hbm.at[pl.ds(r0, 128)], idxb.at[0, 0])
        pltpu.sync_copy(xb.at[0, pl.ds(0, 128)], acc_sh.at[idxb.at[0, 0]],
                        add=True)
        pltpu.sync_copy(onesv, cnt_sh.at[idxb.at[0, 0]], add=True)
        pltpu.sync_copy(b_hbm.at[pl.ds(r0 + 128, 32)], idx_t)
        pltpu.sync_copy(xb.at[0, pl.ds(128, 32)], acc_sh.at[idx_t], add=True)
        pltpu.sync_copy(onesv.at[pl.ds(0, 32)], cnt_sh.at[idx_t], add=True)

    plsc.subcore_barrier()

    pltpu.sync_copy(acc_sh.at[pl.ds(seg0, SEG_PER_TILE)], accv)
    pltpu.sync_copy(cnt_sh.at[pl.ds(seg0, SEG_PER_TILE)], cntv)
    for i in range(SEG_PER_TILE):
        inv = 1.0 / jnp.maximum(cntv[i, :], 1.0)
        for j in range(HALF // 16):
            accv[i, pl.ds(16 * j, 16)] = accv[i, pl.ds(16 * j, 16)] * inv
    pltpu.sync_copy(accv,
                    out_hbm.at[pl.ds(seg0, SEG_PER_TILE), pl.ds(col0, HALF)])


def kernel(x, batch):
    zeros = jnp.zeros((SEG_PER_TILE, HALF), jnp.float32)
    ones = jnp.ones((128, 16), jnp.float32)
    return _pool_kernel(x, batch, zeros, ones)
